# baseline (speedup 1.0000x reference)
import functools

import jax
import jax.numpy as jnp
from jax import lax
from jax.experimental import pallas as pl
from jax.experimental.pallas import tpu as pltpu
from jax.experimental.pallas import tpu_sc as plsc

B = 16
L = 4096
LP1 = L + 1
LANES = 16
NCHUNK = L // LANES
OUTBUF = (NCHUNK + 1) * LANES
START_TOKEN = 1.0
STOP_TOKEN = 2.0

_mesh = plsc.VectorSubcoreMesh(core_axis_name="c", subcore_axis_name="s")


@functools.partial(
    pl.kernel,
    mesh=_mesh,
    out_type=(
        jax.ShapeDtypeStruct((B, LP1), jnp.float32),
        jax.ShapeDtypeStruct((B,), jnp.int32),
        jax.ShapeDtypeStruct((B, LP1), jnp.float32),
        jax.ShapeDtypeStruct((B,), jnp.int32),
    ),
    scratch_types=[
        pltpu.VMEM((L,), jnp.float32),
        pltpu.VMEM((OUTBUF,), jnp.float32),
        pltpu.VMEM((LANES,), jnp.int32),
    ],
    compiler_params=pltpu.CompilerParams(use_tc_tiling_on_sc=False),
)
def _collate(rows_hbm, inp_hbm, inp_len_hbm, tgt_hbm, tgt_len_hbm,
             in_v, out_v, len_v):
    cid = lax.axis_index("c")
    sid = lax.axis_index("s")
    wid = sid * 2 + cid
    b = wid // 2
    which = wid % 2
    row = b * 2 + which

    lane = lax.iota(jnp.int32, LANES)
    rot = jnp.where(lane == 0, LANES - 1, lane - 1)
    rot_idx = rot.reshape(LANES, 1)
    gd = lax.GatherDimensionNumbers(
        offset_dims=(), collapsed_slice_dims=(0,), start_index_map=(0,))

    def rotate1(x):
        return lax.gather(x, rot_idx, gd, (1,),
                          mode=lax.GatherScatterMode.PROMISE_IN_BOUNDS)

    pltpu.sync_copy(rows_hbm.at[row], in_v)

    @pl.when(which == 0)
    def _():
        def body(j, carry):
            cur = in_v[pl.ds(j * LANES, LANES)]
            rcur = rotate1(cur)
            out_v[pl.ds(j * LANES, LANES)] = jnp.where(lane == 0, carry, rcur)
            return rcur

        carry = lax.fori_loop(0, NCHUNK, body,
                              jnp.full((LANES,), START_TOKEN, jnp.float32))
        out_v[pl.ds(L, LANES)] = carry
        pltpu.sync_copy(out_v.at[pl.ds(0, LP1)], inp_hbm.at[b])

    @pl.when(which == 1)
    def _():
        def body(j, _):
            out_v[pl.ds(j * LANES, LANES)] = in_v[pl.ds(j * LANES, LANES)]
            return _

        lax.fori_loop(0, NCHUNK, body, 0)
        out_v[pl.ds(L, LANES)] = jnp.full((LANES,), STOP_TOKEN, jnp.float32)
        pltpu.sync_copy(out_v.at[pl.ds(0, LP1)], tgt_hbm.at[b])

    @pl.when(wid == 0)
    def _():
        len_v[...] = jnp.full((LANES,), LP1, jnp.int32)
        pltpu.sync_copy(len_v, inp_len_hbm)
        pltpu.sync_copy(len_v, tgt_len_hbm)


def kernel(samples):
    rows = samples.reshape(B * 2, L)
    return _collate(rows)


# speedup vs baseline: 10.0934x; 10.0934x over previous
import functools

import jax
import jax.numpy as jnp
from jax.experimental import pallas as pl
from jax.experimental.pallas import tpu as pltpu

B = 16
L = 4096
LP1 = L + 1
START_TOKEN = 1.0
STOP_TOKEN = 2.0

GRID = 2
BB = B // GRID


def _collate_body(x_ref, inp_ref, inp_len_ref, tgt_ref, tgt_len_ref):
    x0 = x_ref[:, 0, :]
    x1 = x_ref[:, 1, :]
    start = jnp.full((BB, 1), START_TOKEN, jnp.float32)
    stop = jnp.full((BB, 1), STOP_TOKEN, jnp.float32)
    inp_ref[...] = jnp.concatenate([start, x0], axis=1)
    tgt_ref[...] = jnp.concatenate([x1, stop], axis=1)
    inp_len_ref[...] = jnp.full((B,), LP1, jnp.int32)
    tgt_len_ref[...] = jnp.full((B,), LP1, jnp.int32)


@jax.jit
def kernel(samples):
    return pl.pallas_call(
        _collate_body,
        grid=(GRID,),
        in_specs=[pl.BlockSpec((BB, 2, L), lambda i: (i, 0, 0))],
        out_specs=(
            pl.BlockSpec((BB, LP1), lambda i: (i, 0)),
            pl.BlockSpec((B,), lambda i: (0,)),
            pl.BlockSpec((BB, LP1), lambda i: (i, 0)),
            pl.BlockSpec((B,), lambda i: (0,)),
        ),
        out_shape=(
            jax.ShapeDtypeStruct((B, LP1), jnp.float32),
            jax.ShapeDtypeStruct((B,), jnp.int32),
            jax.ShapeDtypeStruct((B, LP1), jnp.float32),
            jax.ShapeDtypeStruct((B,), jnp.int32),
        ),
    )(samples)
